# baseline (speedup 1.0000x reference)
import dataclasses
import functools

import jax
import jax.numpy as jnp
from jax import lax
from jax.experimental import pallas as pl
from jax.experimental.pallas import tpu as pltpu
from jax.experimental.pallas import tpu_sc as plsc

N = 10000
E = 320000
CHUNK = 128
NCHUNK = E // CHUNK
NWORK = 32
KMAX = (NCHUNK + NWORK - 1) // NWORK
NPAD = 10240
SUBROWS = NPAD // 16
BLK = 5120
NBLK = NPAD // BLK


def _lrelu_exp(e):
    return jnp.exp(jnp.where(e > 0, e, 0.2 * e))


def _dotT(a, w):
    return lax.dot_general(a, w, (((1,), (1,)), ((), ())),
                           preferred_element_type=jnp.float32)


def _pack_bf16(lo, hi):
    def rne(x):
        u = lax.bitcast_convert_type(x, jnp.int32)
        return (u + 0x7FFF + ((u >> 16) & 1)) >> 16

    return (rne(lo) & 0xFFFF) | (rne(hi) << 16)


def _elu(v):
    return jnp.where(v > 0, v, jnp.exp(jnp.minimum(v, 0.0)) - 1.0)


def _tc1_body(x_ref, w1_ref, b1_ref, g1_ref, be1_ref, w2_ref, b2_ref,
              g2_ref, be2_ref, gw_ref, gas_ref, gad_ref,
              feat_ref, faug_ref, as_ref, ad_ref):
    s1 = g1_ref[...] * (1.0 / jnp.sqrt(1.0 + 1e-3))
    h = _elu(_dotT(x_ref[...], w1_ref[...]) * s1[None, :]
             + (b1_ref[...] * s1 + be1_ref[...])[None, :])
    s2 = g2_ref[...] * (1.0 / jnp.sqrt(1.0 + 1e-3))
    f = _elu(_dotT(h, w2_ref[...]) * s2[None, :]
             + (b2_ref[...] * s2 + be2_ref[...])[None, :])
    feat_ref[...] = f
    lo = jnp.concatenate([f[:, :16]], axis=1)
    hi = jnp.concatenate(
        [f[:, 16:20], jnp.ones((BLK, 1), f.dtype),
         jnp.zeros((BLK, 11), f.dtype)], axis=1)
    faug_ref[...] = _pack_bf16(lo, hi)
    h1 = _dotT(f, gw_ref[...])
    as_ref[...] = h1 @ gas_ref[...]
    ad_ref[...] = h1 @ gad_ref[...]


def _tc1(x, w1, b1, g1, be1, w2, b2, g2, be2, gw, gas, gad):
    f32 = jnp.float32
    full = lambda s: pl.BlockSpec(s, lambda i: tuple(0 for _ in s))
    vec = pl.BlockSpec((BLK,), lambda i: (i,))
    return pl.pallas_call(
        _tc1_body,
        grid=(NBLK,),
        in_specs=[
            pl.BlockSpec((BLK, 128), lambda i: (i, 0)),
            full((32, 128)), full((32,)), full((32,)), full((32,)),
            full((20, 32)), full((20,)), full((20,)), full((20,)),
            full((64, 20)), full((64,)), full((64,)),
        ],
        out_specs=[
            pl.BlockSpec((BLK, 20), lambda i: (i, 0)),
            pl.BlockSpec((BLK, 16), lambda i: (i, 0)),
            vec, vec,
        ],
        out_shape=[
            jax.ShapeDtypeStruct((N, 20), f32),
            jax.ShapeDtypeStruct((NPAD, 16), jnp.int32),
            jax.ShapeDtypeStruct((NPAD,), f32),
            jax.ShapeDtypeStruct((NPAD,), f32),
        ],
    )(x, w1, b1, g1, be1, w2, b2, g2, be2, gw, gas, gad)


def _sc_pass(dual, table, ei3, t0s, t0d, t1s, t1d, zeros):
    f32 = jnp.float32
    i32 = jnp.int32
    mesh = plsc.VectorSubcoreMesh(core_axis_name="c", subcore_axis_name="s")
    KPAD = KMAX + 1
    scratch = [
        pltpu.VMEM((NPAD,), f32),
        pltpu.VMEM((NPAD,), f32),
        pltpu.VMEM((NPAD,), f32),
        pltpu.VMEM((NPAD,), f32),
        pltpu.VMEM((KPAD,), i32),
        pltpu.VMEM((KPAD, CHUNK), i32),
        pltpu.VMEM((KPAD, CHUNK), i32),
        pltpu.VMEM((CHUNK, 16), i32),
        pltpu.VMEM((CHUNK, 16), i32),
        pltpu.VMEM((CHUNK, 32), f32),
        pltpu.VMEM((CHUNK, 32), f32),
        pltpu.VMEM_SHARED((NPAD, 32), f32),
        pltpu.SemaphoreType.DMA,
        pltpu.SemaphoreType.DMA,
        pltpu.SemaphoreType.DMA,
        pltpu.SemaphoreType.DMA,
        pltpu.SemaphoreType.DMA,
    ]

    cp = pltpu.CompilerParams(use_tc_tiling_on_sc=False)
    if "needs_layout_passes" in pltpu.CompilerParams.__dataclass_fields__:
        cp = dataclasses.replace(cp, needs_layout_passes=False)

    @functools.partial(
        pl.kernel,
        out_type=jax.ShapeDtypeStruct((2, NPAD, 32), f32),
        mesh=mesh,
        scratch_types=scratch,
        compiler_params=cp,
    )
    def body(tab_hbm, ei_hbm, a0s_hbm, a0d_hbm, a1s_hbm, a1d_hbm,
             z_hbm, out_hbm, a0s_v, a0d_v, a1s_v, a1d_v, ids_v, src_v, dst_v,
             rows0_v, rows1_v, sca0_v, sca1_v, acc_sh, semi, gsem0, gsem1,
             ssem0, ssem1):
        cid = lax.axis_index("c")
        sid = lax.axis_index("s")
        wid = sid * 2 + cid
        iot = lax.iota(i32, 16)
        base = iot * NWORK + wid
        for j in range(KPAD // 16):
            ids_v[pl.ds(j * 16, 16)] = jnp.minimum(base + j * 16 * NWORK,
                                                   NCHUNK - 1)
        cpy_s = pltpu.async_copy(ei_hbm.at[0].at[ids_v], src_v, semi)
        cpy_d = pltpu.async_copy(ei_hbm.at[1].at[ids_v], dst_v, gsem0)
        pltpu.sync_copy(a0s_hbm, a0s_v)
        pltpu.sync_copy(a0d_hbm, a0d_v)
        if dual:
            pltpu.sync_copy(a1s_hbm, a1s_v)
            pltpu.sync_copy(a1d_hbm, a1d_v)
        pltpu.sync_copy(z_hbm.at[pl.ds(sid * SUBROWS, SUBROWS)],
                        acc_sh.at[pl.ds(sid * SUBROWS, SUBROWS)])
        cpy_s.wait()
        cpy_d.wait()
        plsc.subcore_barrier()

        rows = (rows0_v, rows1_v)
        sca = (sca0_v, sca1_v)
        gsem = (gsem0, gsem1)
        ssem = (ssem0, ssem1)
        pltpu.async_copy(tab_hbm.at[src_v.at[0]], rows0_v, gsem0)

        def process(k, b):
            rows_v = rows[b]
            sca_v = sca[b]
            pltpu.make_async_copy(tab_hbm.at[src_v.at[0]], rows_v,
                                  gsem[b]).wait()

            @pl.when(k + 1 < KMAX)
            def _():
                pltpu.async_copy(tab_hbm.at[src_v.at[k + 1]], rows[1 - b],
                                 gsem[1 - b])

            @pl.when((k >= 2) & ((k - 2) * NWORK + wid < NCHUNK))
            def _():
                pltpu.make_async_copy(sca_v, acc_sh.at[dst_v.at[0]],
                                      ssem[b]).wait()

            for j in range(CHUNK // 16):
                sv = src_v[k, pl.ds(j * 16, 16)]
                dv = dst_v[k, pl.ds(j * 16, 16)]
                wv0 = _lrelu_exp(plsc.load_gather(a0s_v, [sv])
                                 + plsc.load_gather(a0d_v, [dv]))
                wv1 = (_lrelu_exp(plsc.load_gather(a1s_v, [sv])
                                  + plsc.load_gather(a1d_v, [dv]))
                       if dual else wv0)
                for l in range(16):
                    i = j * 16 + l
                    u = rows_v[i, pl.ds(0, 16)]
                    lo = plsc.bitcast(u << 16, f32)
                    hi = plsc.bitcast(u & jnp.int32(-65536), f32)
                    sca_v[i, pl.ds(0, 16)] = lo * wv0[l]
                    sca_v[i, pl.ds(16, 16)] = hi * wv1[l]

            @pl.when(k * NWORK + wid < NCHUNK)
            def _():
                pltpu.async_copy(sca_v, acc_sh.at[dst_v.at[k]], ssem[b],
                                 add=True)

        @pl.loop(0, KPAD // 2)
        def _(i):
            for b in range(2):
                k = i * 2 + b

                @pl.when(k < KMAX)
                def _():
                    process(k, b)

        @pl.when((KMAX - 2) * NWORK + wid < NCHUNK)
        def _():
            pltpu.make_async_copy(sca[(KMAX - 2) % 2],
                                  acc_sh.at[dst_v.at[0]],
                                  ssem[(KMAX - 2) % 2]).wait()

        @pl.when((KMAX - 1) * NWORK + wid < NCHUNK)
        def _():
            pltpu.make_async_copy(sca[(KMAX - 1) % 2],
                                  acc_sh.at[dst_v.at[0]],
                                  ssem[(KMAX - 1) % 2]).wait()

        plsc.subcore_barrier()
        pltpu.sync_copy(acc_sh.at[pl.ds(sid * SUBROWS, SUBROWS)],
                        out_hbm.at[cid, pl.ds(sid * SUBROWS, SUBROWS)])

    return body(table, ei3, t0s, t0d, t1s, t1d, zeros)


def _tc2_body(acc_ref, feat_ref, as_ref, ad_ref, gw_ref, gb_ref, bg_ref,
              bb_ref, gmu_ref, gmas_ref, gmad_ref, glv_ref, glas_ref,
              glad_ref,
              g2_ref, amus_ref, amud_ref, alvs_ref, alvd_ref):
    s1 = _lrelu_exp(as_ref[...] + ad_ref[...])
    num = acc_ref[0, :, :20] + acc_ref[1, :, :20] + s1[:, None] * feat_ref[...]
    den = acc_ref[0, :, 20] + acc_ref[1, :, 20] + s1
    sg = bg_ref[...] * (1.0 / jnp.sqrt(1.0 + 1e-5))
    c = jnp.maximum(_dotT(num / den[:, None], gw_ref[...]) * sg[None, :]
                    + (gb_ref[...] * sg + bb_ref[...])[None, :], 0.0)
    hmu = _dotT(c, gmu_ref[...])
    hlv = _dotT(c, glv_ref[...])
    one = jnp.ones((BLK, 1), c.dtype)
    z7 = jnp.zeros((BLK, 7), c.dtype)
    g2_ref[...] = _pack_bf16(jnp.concatenate([hmu, one, z7], axis=1),
                             jnp.concatenate([hlv, one, z7], axis=1))
    amus_ref[...] = hmu @ gmas_ref[...]
    amud_ref[...] = hmu @ gmad_ref[...]
    alvs_ref[...] = hlv @ glas_ref[...]
    alvd_ref[...] = hlv @ glad_ref[...]


def _tc2(acc1, feat, as1, ad1, gw, gb, bg, bb, gmu, gmas, gmad, glv, glas,
         glad):
    f32 = jnp.float32
    full = lambda s: pl.BlockSpec(s, lambda i: tuple(0 for _ in s))
    vec = pl.BlockSpec((BLK,), lambda i: (i,))
    vshape = jax.ShapeDtypeStruct((NPAD,), f32)
    return pl.pallas_call(
        _tc2_body,
        grid=(NBLK,),
        in_specs=[
            pl.BlockSpec((2, BLK, 32), lambda i: (0, i, 0)),
            pl.BlockSpec((BLK, 20), lambda i: (i, 0)),
            vec, vec,
            full((64, 20)), full((64,)), full((64,)), full((64,)),
            full((8, 64)), full((8,)), full((8,)),
            full((8, 64)), full((8,)), full((8,)),
        ],
        out_specs=[
            pl.BlockSpec((BLK, 16), lambda i: (i, 0)),
            vec, vec, vec, vec,
        ],
        out_shape=[
            jax.ShapeDtypeStruct((NPAD, 16), jnp.int32),
            vshape, vshape, vshape, vshape,
        ],
    )(acc1, feat, as1, ad1, gw, gb, bg, bb, gmu, gmas, gmad, glv, glas, glad)


def _tc3_body(acc_ref, g2_ref, amus_ref, amud_ref, alvs_ref, alvd_ref,
              feat_ref, gmub_ref, glvb_ref, wd1_ref, bd1_ref, gd1_ref,
              bed1_ref, wdo_ref, bdo_ref, clus_ref,
              z_ref, mu_ref, lv_ref, dft_ref, q_ref):
    smu = _lrelu_exp(amus_ref[...] + amud_ref[...])
    slv = _lrelu_exp(alvs_ref[...] + alvd_ref[...])
    g2p = g2_ref[...][:, 0:8]
    hmu = lax.bitcast_convert_type(g2p << 16, jnp.float32)
    hlv = lax.bitcast_convert_type(g2p & jnp.int32(-65536), jnp.float32)
    num_mu = acc_ref[0, :, 0:8] + acc_ref[1, :, 0:8] + smu[:, None] * hmu
    den_mu = acc_ref[0, :, 8] + acc_ref[1, :, 8] + smu
    mu = num_mu / den_mu[:, None] + gmub_ref[...][None, :]
    num_lv = (acc_ref[0, :, 16:24] + acc_ref[1, :, 16:24]
              + slv[:, None] * hlv)
    den_lv = acc_ref[0, :, 24] + acc_ref[1, :, 24] + slv
    lv = num_lv / den_lv[:, None] + glvb_ref[...][None, :]
    z = jnp.concatenate([feat_ref[...], mu], axis=1)
    z_ref[...] = z
    mu_ref[...] = mu
    lv_ref[...] = lv
    sd = gd1_ref[...] * (1.0 / jnp.sqrt(1.0 + 1e-3))
    d = _elu(_dotT(z, wd1_ref[...]) * sd[None, :]
             + (bd1_ref[...] * sd + bed1_ref[...])[None, :])
    dft_ref[...] = _dotT(d, wdo_ref[...]) + bdo_ref[...][None, :]
    cl = clus_ref[...]
    sq = (jnp.sum(z * z, axis=1)[:, None] - 2.0 * _dotT(z, cl)
          + jnp.sum(cl * cl, axis=1)[None, :])
    u = 1.0 / (1.0 + sq / 0.9)
    q = jnp.exp(0.95 * jnp.log(u))
    q_ref[...] = q / jnp.sum(q, axis=1, keepdims=True)


def _tc3(acc2, g2, amus, amud, alvs, alvd, feat, gmub, glvb, wd1, bd1,
         gd1, bed1, wdo, bdo, clus):
    f32 = jnp.float32
    full = lambda s: pl.BlockSpec(s, lambda i: tuple(0 for _ in s))
    vec = pl.BlockSpec((BLK,), lambda i: (i,))
    return pl.pallas_call(
        _tc3_body,
        grid=(NBLK,),
        in_specs=[
            pl.BlockSpec((2, BLK, 32), lambda i: (0, i, 0)),
            pl.BlockSpec((BLK, 16), lambda i: (i, 0)),
            vec, vec, vec, vec,
            pl.BlockSpec((BLK, 20), lambda i: (i, 0)),
            full((8,)), full((8,)), full((32, 28)), full((32,)),
            full((32,)), full((32,)), full((128, 32)), full((128,)),
            full((15, 28)),
        ],
        out_specs=[
            pl.BlockSpec((BLK, 28), lambda i: (i, 0)),
            pl.BlockSpec((BLK, 8), lambda i: (i, 0)),
            pl.BlockSpec((BLK, 8), lambda i: (i, 0)),
            pl.BlockSpec((BLK, 128), lambda i: (i, 0)),
            pl.BlockSpec((BLK, 15), lambda i: (i, 0)),
        ],
        out_shape=[
            jax.ShapeDtypeStruct((N, 28), f32),
            jax.ShapeDtypeStruct((N, 8), f32),
            jax.ShapeDtypeStruct((N, 8), f32),
            jax.ShapeDtypeStruct((N, 128), f32),
            jax.ShapeDtypeStruct((N, 15), f32),
        ],
    )(acc2, g2, amus, amud, alvs, alvd, feat, gmub, glvb, wd1, bd1,
      gd1, bed1, wdo, bdo, clus)


def kernel(x, edge_index, enc_W1, enc_b1, enc_g1, enc_be1, enc_W2, enc_b2,
           enc_g2, enc_be2, gat1_W, gat1_as, gat1_ad, gat1_b, bn1_g, bn1_b,
           gmu_W, gmu_as, gmu_ad, gmu_b, glv_W, glv_as, glv_ad, glv_b,
           dec_W1, dec_b1, dec_g1, dec_be1, dec_Wout, dec_bout, cluster):
    f32 = jnp.float32
    ei3 = edge_index.reshape(2, NCHUNK, CHUNK)
    zeros = jnp.zeros((NPAD, 32), f32)

    feat, faug, as1, ad1 = _tc1(x, enc_W1, enc_b1, enc_g1, enc_be1,
                                enc_W2, enc_b2, enc_g2, enc_be2,
                                gat1_W, gat1_as, gat1_ad)
    acc1 = _sc_pass(False, faug, ei3, as1, ad1, as1, ad1, zeros)
    g2, amus, amud, alvs, alvd = _tc2(acc1, feat, as1, ad1, gat1_W, gat1_b,
                                      bn1_g, bn1_b, gmu_W, gmu_as, gmu_ad,
                                      glv_W, glv_as, glv_ad)
    acc2 = _sc_pass(True, g2, ei3, amus, amud, alvs, alvd, zeros)
    z, mu, lv, dft, q = _tc3(acc2, g2, amus, amud, alvs, alvd, feat,
                             gmu_b, glv_b, dec_W1, dec_b1, dec_g1, dec_be1,
                             dec_Wout, dec_bout, cluster)
    return (z, mu, lv, dft, q, feat, mu)


# speedup vs baseline: 1.1951x; 1.1951x over previous
import dataclasses
import functools

import jax
import jax.numpy as jnp
from jax import lax
from jax.experimental import pallas as pl
from jax.experimental.pallas import tpu as pltpu
from jax.experimental.pallas import tpu_sc as plsc

N = 10000
E = 320000
CHUNK = 128
NCHUNK = E // CHUNK
NWORK = 32
KMAX = (NCHUNK + NWORK - 1) // NWORK
NPAD = 10240
SUBROWS = NPAD // 16
BLK = 5120
NBLK = NPAD // BLK


def _lrelu_exp(e):
    return jnp.exp(jnp.where(e > 0, e, 0.2 * e))


def _dotT(a, w):
    return lax.dot_general(a, w, (((1,), (1,)), ((), ())),
                           preferred_element_type=jnp.float32)


def _elu(v):
    return jnp.where(v > 0, v, jnp.exp(jnp.minimum(v, 0.0)) - 1.0)


def _tc1_body(x_ref, w1_ref, b1_ref, g1_ref, be1_ref, w2_ref, b2_ref,
              g2_ref, be2_ref, gw_ref, gas_ref, gad_ref,
              feat_ref, faug_ref, as_ref, ad_ref):
    s1 = g1_ref[...] * (1.0 / jnp.sqrt(1.0 + 1e-3))
    h = _elu(_dotT(x_ref[...], w1_ref[...]) * s1[None, :]
             + (b1_ref[...] * s1 + be1_ref[...])[None, :])
    s2 = g2_ref[...] * (1.0 / jnp.sqrt(1.0 + 1e-3))
    f = _elu(_dotT(h, w2_ref[...]) * s2[None, :]
             + (b2_ref[...] * s2 + be2_ref[...])[None, :])
    feat_ref[...] = f
    faug_ref[...] = jnp.concatenate(
        [f, jnp.ones((BLK, 1), f.dtype), jnp.zeros((BLK, 11), f.dtype)], axis=1)
    h1 = _dotT(f, gw_ref[...])
    as_ref[...] = h1 @ gas_ref[...]
    ad_ref[...] = h1 @ gad_ref[...]


def _tc1(x, w1, b1, g1, be1, w2, b2, g2, be2, gw, gas, gad):
    f32 = jnp.float32
    full = lambda s: pl.BlockSpec(s, lambda i: tuple(0 for _ in s))
    vec = pl.BlockSpec((BLK,), lambda i: (i,))
    return pl.pallas_call(
        _tc1_body,
        grid=(NBLK,),
        in_specs=[
            pl.BlockSpec((BLK, 128), lambda i: (i, 0)),
            full((32, 128)), full((32,)), full((32,)), full((32,)),
            full((20, 32)), full((20,)), full((20,)), full((20,)),
            full((64, 20)), full((64,)), full((64,)),
        ],
        out_specs=[
            pl.BlockSpec((BLK, 20), lambda i: (i, 0)),
            pl.BlockSpec((BLK, 32), lambda i: (i, 0)),
            vec, vec,
        ],
        out_shape=[
            jax.ShapeDtypeStruct((N, 20), f32),
            jax.ShapeDtypeStruct((NPAD, 32), f32),
            jax.ShapeDtypeStruct((NPAD,), f32),
            jax.ShapeDtypeStruct((NPAD,), f32),
        ],
    )(x, w1, b1, g1, be1, w2, b2, g2, be2, gw, gas, gad)


def _sc_pass(dual, table, ei3, t0s, t0d, t1s, t1d, zeros):
    f32 = jnp.float32
    i32 = jnp.int32
    mesh = plsc.VectorSubcoreMesh(core_axis_name="c", subcore_axis_name="s")
    KPAD = KMAX + 1
    scratch = [
        pltpu.VMEM((NPAD,), f32),
        pltpu.VMEM((NPAD,), f32),
        pltpu.VMEM((NPAD,), f32),
        pltpu.VMEM((NPAD,), f32),
        pltpu.VMEM((KPAD,), i32),
        pltpu.VMEM((KPAD, CHUNK), i32),
        pltpu.VMEM((KPAD, CHUNK), i32),
        pltpu.VMEM((CHUNK, 32), f32),
        pltpu.VMEM((CHUNK, 32), f32),
        pltpu.VMEM_SHARED((NPAD, 32), f32),
        pltpu.VMEM_SHARED((NPAD, 32), f32),
        pltpu.SemaphoreType.DMA,
        pltpu.SemaphoreType.DMA,
        pltpu.SemaphoreType.DMA,
    ]

    cp = pltpu.CompilerParams(use_tc_tiling_on_sc=False)
    if "needs_layout_passes" in pltpu.CompilerParams.__dataclass_fields__:
        cp = dataclasses.replace(cp, needs_layout_passes=False)

    @functools.partial(
        pl.kernel,
        out_type=jax.ShapeDtypeStruct((2, NPAD, 32), f32),
        mesh=mesh,
        scratch_types=scratch,
        compiler_params=cp,
    )
    def body(tab_hbm, ei_hbm, a0s_hbm, a0d_hbm, a1s_hbm, a1d_hbm,
             z_hbm, out_hbm, a0s_v, a0d_v, a1s_v, a1d_v, ids_v, src_v, dst_v,
             rows0_v, rows1_v, acc_sh, tab_sh, semi, gsem0, gsem1):
        cid = lax.axis_index("c")
        sid = lax.axis_index("s")
        wid = sid * 2 + cid
        iot = lax.iota(i32, 16)
        base = iot * NWORK + wid
        for j in range(KPAD // 16):
            ids_v[pl.ds(j * 16, 16)] = jnp.minimum(base + j * 16 * NWORK,
                                                   NCHUNK - 1)
        cpy_s = pltpu.async_copy(ei_hbm.at[0].at[ids_v], src_v, semi)
        cpy_d = pltpu.async_copy(ei_hbm.at[1].at[ids_v], dst_v, gsem0)
        pltpu.sync_copy(a0s_hbm, a0s_v)
        pltpu.sync_copy(a0d_hbm, a0d_v)
        if dual:
            pltpu.sync_copy(a1s_hbm, a1s_v)
            pltpu.sync_copy(a1d_hbm, a1d_v)
        pltpu.sync_copy(z_hbm.at[pl.ds(sid * SUBROWS, SUBROWS)],
                        acc_sh.at[pl.ds(sid * SUBROWS, SUBROWS)])
        pltpu.sync_copy(tab_hbm.at[pl.ds(sid * SUBROWS, SUBROWS)],
                        tab_sh.at[pl.ds(sid * SUBROWS, SUBROWS)])
        cpy_s.wait()
        cpy_d.wait()
        plsc.subcore_barrier()

        rows = (rows0_v, rows1_v)
        gsem = (gsem0, gsem1)
        pltpu.async_copy(tab_sh.at[src_v.at[0]], rows0_v, gsem0)

        def process(k, b):
            rows_v = rows[b]
            pltpu.make_async_copy(tab_sh.at[src_v.at[0]], rows_v,
                                  gsem[b]).wait()

            @pl.when(k + 1 < KMAX)
            def _():
                pltpu.async_copy(tab_sh.at[src_v.at[k + 1]], rows[1 - b],
                                 gsem[1 - b])

            for j in range(CHUNK // 16):
                sv = src_v[k, pl.ds(j * 16, 16)]
                dv = dst_v[k, pl.ds(j * 16, 16)]
                wv0 = _lrelu_exp(plsc.load_gather(a0s_v, [sv])
                                 + plsc.load_gather(a0d_v, [dv]))
                wv1 = (_lrelu_exp(plsc.load_gather(a1s_v, [sv])
                                  + plsc.load_gather(a1d_v, [dv]))
                       if dual else wv0)
                for l in range(16):
                    i = j * 16 + l
                    rows_v[i, pl.ds(0, 16)] = rows_v[i, pl.ds(0, 16)] * wv0[l]
                    rows_v[i, pl.ds(16, 16)] = rows_v[i, pl.ds(16, 16)] * wv1[l]

            @pl.when(k * NWORK + wid < NCHUNK)
            def _():
                pltpu.sync_copy(rows_v, acc_sh.at[dst_v.at[k]], add=True)

        @pl.loop(0, KPAD // 2)
        def _(i):
            for b in range(2):
                k = i * 2 + b

                @pl.when(k < KMAX)
                def _():
                    process(k, b)

        plsc.subcore_barrier()
        pltpu.sync_copy(acc_sh.at[pl.ds(sid * SUBROWS, SUBROWS)],
                        out_hbm.at[cid, pl.ds(sid * SUBROWS, SUBROWS)])

    return body(table, ei3, t0s, t0d, t1s, t1d, zeros)


def _tc2_body(acc_ref, feat_ref, as_ref, ad_ref, gw_ref, gb_ref, bg_ref,
              bb_ref, gmu_ref, gmas_ref, gmad_ref, glv_ref, glas_ref,
              glad_ref,
              g2_ref, amus_ref, amud_ref, alvs_ref, alvd_ref):
    s1 = _lrelu_exp(as_ref[...] + ad_ref[...])
    num = acc_ref[0, :, :20] + acc_ref[1, :, :20] + s1[:, None] * feat_ref[...]
    den = acc_ref[0, :, 20] + acc_ref[1, :, 20] + s1
    sg = bg_ref[...] * (1.0 / jnp.sqrt(1.0 + 1e-5))
    c = jnp.maximum(_dotT(num / den[:, None], gw_ref[...]) * sg[None, :]
                    + (gb_ref[...] * sg + bb_ref[...])[None, :], 0.0)
    hmu = _dotT(c, gmu_ref[...])
    hlv = _dotT(c, glv_ref[...])
    one = jnp.ones((BLK, 1), c.dtype)
    z7 = jnp.zeros((BLK, 7), c.dtype)
    g2_ref[...] = jnp.concatenate([hmu, one, z7, hlv, one, z7], axis=1)
    amus_ref[...] = hmu @ gmas_ref[...]
    amud_ref[...] = hmu @ gmad_ref[...]
    alvs_ref[...] = hlv @ glas_ref[...]
    alvd_ref[...] = hlv @ glad_ref[...]


def _tc2(acc1, feat, as1, ad1, gw, gb, bg, bb, gmu, gmas, gmad, glv, glas,
         glad):
    f32 = jnp.float32
    full = lambda s: pl.BlockSpec(s, lambda i: tuple(0 for _ in s))
    vec = pl.BlockSpec((BLK,), lambda i: (i,))
    vshape = jax.ShapeDtypeStruct((NPAD,), f32)
    return pl.pallas_call(
        _tc2_body,
        grid=(NBLK,),
        in_specs=[
            pl.BlockSpec((2, BLK, 32), lambda i: (0, i, 0)),
            pl.BlockSpec((BLK, 20), lambda i: (i, 0)),
            vec, vec,
            full((64, 20)), full((64,)), full((64,)), full((64,)),
            full((8, 64)), full((8,)), full((8,)),
            full((8, 64)), full((8,)), full((8,)),
        ],
        out_specs=[
            pl.BlockSpec((BLK, 32), lambda i: (i, 0)),
            vec, vec, vec, vec,
        ],
        out_shape=[
            jax.ShapeDtypeStruct((NPAD, 32), f32),
            vshape, vshape, vshape, vshape,
        ],
    )(acc1, feat, as1, ad1, gw, gb, bg, bb, gmu, gmas, gmad, glv, glas, glad)


def _tc3_body(acc_ref, g2_ref, amus_ref, amud_ref, alvs_ref, alvd_ref,
              feat_ref, gmub_ref, glvb_ref, wd1_ref, bd1_ref, gd1_ref,
              bed1_ref, wdo_ref, bdo_ref, clus_ref,
              z_ref, mu_ref, lv_ref, dft_ref, q_ref):
    smu = _lrelu_exp(amus_ref[...] + amud_ref[...])
    slv = _lrelu_exp(alvs_ref[...] + alvd_ref[...])
    g2 = g2_ref[...]
    hmu = g2[:, 0:8]
    hlv = g2[:, 16:24]
    num_mu = acc_ref[0, :, 0:8] + acc_ref[1, :, 0:8] + smu[:, None] * hmu
    den_mu = acc_ref[0, :, 8] + acc_ref[1, :, 8] + smu
    mu = num_mu / den_mu[:, None] + gmub_ref[...][None, :]
    num_lv = (acc_ref[0, :, 16:24] + acc_ref[1, :, 16:24]
              + slv[:, None] * hlv)
    den_lv = acc_ref[0, :, 24] + acc_ref[1, :, 24] + slv
    lv = num_lv / den_lv[:, None] + glvb_ref[...][None, :]
    z = jnp.concatenate([feat_ref[...], mu], axis=1)
    z_ref[...] = z
    mu_ref[...] = mu
    lv_ref[...] = lv
    sd = gd1_ref[...] * (1.0 / jnp.sqrt(1.0 + 1e-3))
    d = _elu(_dotT(z, wd1_ref[...]) * sd[None, :]
             + (bd1_ref[...] * sd + bed1_ref[...])[None, :])
    dft_ref[...] = _dotT(d, wdo_ref[...]) + bdo_ref[...][None, :]
    cl = clus_ref[...]
    sq = (jnp.sum(z * z, axis=1)[:, None] - 2.0 * _dotT(z, cl)
          + jnp.sum(cl * cl, axis=1)[None, :])
    u = 1.0 / (1.0 + sq / 0.9)
    q = jnp.exp(0.95 * jnp.log(u))
    q_ref[...] = q / jnp.sum(q, axis=1, keepdims=True)


def _tc3(acc2, g2, amus, amud, alvs, alvd, feat, gmub, glvb, wd1, bd1,
         gd1, bed1, wdo, bdo, clus):
    f32 = jnp.float32
    full = lambda s: pl.BlockSpec(s, lambda i: tuple(0 for _ in s))
    vec = pl.BlockSpec((BLK,), lambda i: (i,))
    return pl.pallas_call(
        _tc3_body,
        grid=(NBLK,),
        in_specs=[
            pl.BlockSpec((2, BLK, 32), lambda i: (0, i, 0)),
            pl.BlockSpec((BLK, 32), lambda i: (i, 0)),
            vec, vec, vec, vec,
            pl.BlockSpec((BLK, 20), lambda i: (i, 0)),
            full((8,)), full((8,)), full((32, 28)), full((32,)),
            full((32,)), full((32,)), full((128, 32)), full((128,)),
            full((15, 28)),
        ],
        out_specs=[
            pl.BlockSpec((BLK, 28), lambda i: (i, 0)),
            pl.BlockSpec((BLK, 8), lambda i: (i, 0)),
            pl.BlockSpec((BLK, 8), lambda i: (i, 0)),
            pl.BlockSpec((BLK, 128), lambda i: (i, 0)),
            pl.BlockSpec((BLK, 15), lambda i: (i, 0)),
        ],
        out_shape=[
            jax.ShapeDtypeStruct((N, 28), f32),
            jax.ShapeDtypeStruct((N, 8), f32),
            jax.ShapeDtypeStruct((N, 8), f32),
            jax.ShapeDtypeStruct((N, 128), f32),
            jax.ShapeDtypeStruct((N, 15), f32),
        ],
    )(acc2, g2, amus, amud, alvs, alvd, feat, gmub, glvb, wd1, bd1,
      gd1, bed1, wdo, bdo, clus)


def kernel(x, edge_index, enc_W1, enc_b1, enc_g1, enc_be1, enc_W2, enc_b2,
           enc_g2, enc_be2, gat1_W, gat1_as, gat1_ad, gat1_b, bn1_g, bn1_b,
           gmu_W, gmu_as, gmu_ad, gmu_b, glv_W, glv_as, glv_ad, glv_b,
           dec_W1, dec_b1, dec_g1, dec_be1, dec_Wout, dec_bout, cluster):
    f32 = jnp.float32
    ei3 = edge_index.reshape(2, NCHUNK, CHUNK)
    zeros = jnp.zeros((NPAD, 32), f32)

    feat, faug, as1, ad1 = _tc1(x, enc_W1, enc_b1, enc_g1, enc_be1,
                                enc_W2, enc_b2, enc_g2, enc_be2,
                                gat1_W, gat1_as, gat1_ad)
    acc1 = _sc_pass(False, faug, ei3, as1, ad1, as1, ad1, zeros)
    g2, amus, amud, alvs, alvd = _tc2(acc1, feat, as1, ad1, gat1_W, gat1_b,
                                      bn1_g, bn1_b, gmu_W, gmu_as, gmu_ad,
                                      glv_W, glv_as, glv_ad)
    acc2 = _sc_pass(True, g2, ei3, amus, amud, alvs, alvd, zeros)
    z, mu, lv, dft, q = _tc3(acc2, g2, amus, amud, alvs, alvd, feat,
                             gmu_b, glv_b, dec_W1, dec_b1, dec_g1, dec_be1,
                             dec_Wout, dec_bout, cluster)
    return (z, mu, lv, dft, q, feat, mu)
